# Initial kernel scaffold; baseline (speedup 1.0000x reference)
import jax
import jax.numpy as jnp
from jax.experimental import pallas as pl


def kernel(kpts, scores, dispersity):
    raise NotImplementedError("write your pallas kernel here")


# speedup vs baseline: 1.3699x; 1.3699x over previous
import functools

import jax
import jax.numpy as jnp
from jax.experimental import pallas as pl
from jax.experimental.pallas import tpu as pltpu

_RADIUS2 = 4.0
_SCORES_TH = 0.1
_W = 639.0
_H = 479.0
_N = 5000
_NPAD = 5120
_BI = 512
_BJ = 512
_NEG = -3.0e38


def _nms_body(xr, yr, sr, dr, xc, yc, sc, out, *, n_jblk):
    i = pl.program_id(0)
    xi = xr[...]
    yi = yr[...]
    si = sr[...]

    def jstep(j, acc):
        xj = xc[:, pl.ds(j * _BJ, _BJ)]
        yj = yc[:, pl.ds(j * _BJ, _BJ)]
        sj = sc[:, pl.ds(j * _BJ, _BJ)]
        dx = xi - xj
        dy = yi - yj
        d2 = dx * dx + dy * dy
        ns = jnp.where(d2 < _RADIUS2, sj, _NEG)
        return jnp.maximum(acc, jnp.max(ns, axis=1, keepdims=True))

    acc0 = jnp.full((_BI, 1), _NEG, dtype=jnp.float32)
    neigh_max = jax.lax.fori_loop(0, n_jblk, jstep, acc0)
    valid = jnp.logical_and(si >= neigh_max, si > _SCORES_TH)
    psum = jnp.sum(jnp.where(valid, dr[...], 0.0))
    pcnt = jnp.sum(valid.astype(jnp.float32))

    @pl.when(i == 0)
    def _():
        out[0] = 0.0
        out[1] = 0.0

    out[0] += psum
    out[1] += pcnt


def kernel(kpts, scores, dispersity):
    x = kpts[:, 0] * _W
    y = kpts[:, 1] * _H
    pad = _NPAD - _N
    x = jnp.concatenate([x, jnp.full((pad,), 1.0e6, jnp.float32)])
    y = jnp.concatenate([y, jnp.full((pad,), 1.0e6, jnp.float32)])
    s = jnp.concatenate([scores, jnp.full((pad,), -1.0, jnp.float32)])
    d = jnp.concatenate([dispersity, jnp.zeros((pad,), jnp.float32)])

    xr = x.reshape(_NPAD, 1)
    yr = y.reshape(_NPAD, 1)
    sr = s.reshape(_NPAD, 1)
    dr = d.reshape(_NPAD, 1)
    xc = x.reshape(1, _NPAD)
    yc = y.reshape(1, _NPAD)
    sc = s.reshape(1, _NPAD)

    grid = _NPAD // _BI
    row_spec = pl.BlockSpec((_BI, 1), lambda i: (i, 0))
    col_spec = pl.BlockSpec((1, _NPAD), lambda i: (0, 0))
    out = pl.pallas_call(
        functools.partial(_nms_body, n_jblk=_NPAD // _BJ),
        grid=(grid,),
        in_specs=[row_spec, row_spec, row_spec, row_spec,
                  col_spec, col_spec, col_spec],
        out_specs=pl.BlockSpec(memory_space=pltpu.SMEM),
        out_shape=jax.ShapeDtypeStruct((2,), jnp.float32),
    )(xr, yr, sr, dr, xc, yc, sc)
    loss_sum, cnt = out[0], out[1]
    return jnp.where(cnt > 0, loss_sum / jnp.maximum(cnt, 1.0), jnp.float32(0.0))


# speedup vs baseline: 1.8219x; 1.3299x over previous
import jax
import jax.numpy as jnp
from jax import lax
from jax.experimental import pallas as pl
from jax.experimental.pallas import tpu as pltpu
from jax.experimental.pallas import tpu_sc as plsc

_RADIUS2 = 4.0
_SCORES_TH = 0.1
_W = 639.0
_H = 479.0
_N = 5000
_NW = 16
_NPAD = 5120
_PW = _NPAD // _NW
_L = 16
_NSTR = 324
_SSZ = 336
_CSZ = 352
_SCHUNK = 80
_NEG = -3.0e38
_PADX = 1.0e6


def _sc_body(xh, yh, sh, dh, out_sum, out_cnt,
             xv, yv, sv, dv, sidv, occv, lastv, posv1, posv2,
             cnt, allcnt, totv, wpartv, Cv, shuf,
             sx, sy, ss, sdv, psumr, pcntr,
             shared_cnt, shared_sx, shared_sy, shared_ss, shared_sd):
    wid = lax.axis_index("s")
    base = wid * _PW
    lane = lax.iota(jnp.int32, _L)
    nvec = _PW // _L
    ones_i = jnp.ones((_L,), jnp.int32)
    zeros_i = jnp.zeros((_L,), jnp.int32)

    pltpu.sync_copy(xh.at[pl.ds(base, _PW)], xv)
    pltpu.sync_copy(yh.at[pl.ds(base, _PW)], yv)
    pltpu.sync_copy(sh.at[pl.ds(base, _PW)], sv)
    pltpu.sync_copy(dh.at[pl.ds(base, _PW)], dv)

    def sid_step(k, c):
        sl = pl.ds(k * _L, _L)
        sidv[sl] = jnp.minimum((xv[sl] * 0.5).astype(jnp.int32), _NSTR - 1)
        return c

    lax.fori_loop(0, nvec, sid_step, 0)

    for k in range(_SSZ // _L):
        cnt[pl.ds(k * _L, _L)] = zeros_i

    def count_step(k, c):
        sl = pl.ds(k * _L, _L)
        sid = sidv[sl]

        def shift_step(s, oc_fw):
            occ, fwd = oc_fw
            sb = plsc.load_gather(sidv, [k * _L + jnp.maximum(lane - s, 0)])
            sf = plsc.load_gather(sidv,
                                  [k * _L + jnp.minimum(lane + s, _L - 1)])
            occ = occ + ((lane >= s) & (sb == sid)).astype(jnp.int32)
            fwd = fwd + ((lane + s < _L) & (sf == sid)).astype(jnp.int32)
            return occ, fwd

        occ, fwd = lax.fori_loop(1, _L, shift_step, (ones_i, zeros_i))
        last = fwd == 0
        occv[sl] = occ
        lastv[sl] = last.astype(jnp.int32)
        cur = plsc.load_gather(cnt, [sid])
        plsc.store_scatter(cnt, [sid], cur + occ, mask=last)
        return c

    lax.fori_loop(0, nvec, count_step, 0)

    pltpu.sync_copy(cnt, shared_cnt.at[wid])
    plsc.subcore_barrier()
    pltpu.sync_copy(shared_cnt, allcnt)

    for k in range(_SSZ // _L):
        sl = pl.ds(k * _L, _L)
        tot = jnp.zeros((_L,), jnp.int32)
        part = jnp.zeros((_L,), jnp.int32)
        for w in range(_NW):
            row = allcnt[w, sl]
            tot = tot + row
            part = part + row * (jnp.int32(w) < wid).astype(jnp.int32)
        totv[sl] = tot
        wpartv[sl] = part

    npad_i = jnp.full((_L,), _NPAD, jnp.int32)
    for k in range(_SSZ // _L, _CSZ // _L):
        Cv[pl.ds(k * _L, _L)] = npad_i

    def cum_step(k, carry):
        sl = pl.ds(k * _L, _L)
        v = totv[sl]
        p = v
        for s in (1, 2, 4, 8):
            shuf[...] = p
            g = plsc.load_gather(shuf, [jnp.maximum(lane - s, 0)])
            p = p + g * (lane >= s).astype(jnp.int32)
        Cv[sl] = p - v + carry
        return carry + p[_L - 1]

    lax.fori_loop(0, _SSZ // _L, cum_step, jnp.int32(0))

    def curs_step(k, c):
        sl = pl.ds(k * _L, _L)
        wpartv[sl] = Cv[sl] + wpartv[sl]
        return c

    lax.fori_loop(0, _SSZ // _L, curs_step, 0)

    def place_step(k, c):
        sl = pl.ds(k * _L, _L)
        sid = sidv[sl]
        occ = occv[sl]
        last = lastv[sl] == 1
        b = plsc.load_gather(wpartv, [sid])
        posv1[sl] = b + occ - 1
        plsc.store_scatter(wpartv, [sid], b + occ, mask=last)
        return c

    lax.fori_loop(0, nvec, place_step, 0)

    for k in range(nvec):
        posv2[k // (_SCHUNK // _L),
              pl.ds((k % (_SCHUNK // _L)) * _L, _L)] = posv1[pl.ds(k * _L, _L)]

    for c in range(_PW // _SCHUNK):
        sl = pl.ds(c * _SCHUNK, _SCHUNK)
        idx = posv2.at[c]
        pltpu.sync_copy(xv.at[sl], shared_sx.at[idx])
        pltpu.sync_copy(yv.at[sl], shared_sy.at[idx])
        pltpu.sync_copy(sv.at[sl], shared_ss.at[idx])
        pltpu.sync_copy(dv.at[sl], shared_sd.at[idx])
    plsc.subcore_barrier()

    pltpu.sync_copy(shared_sx, sx)
    pltpu.sync_copy(shared_sy, sy)
    pltpu.sync_copy(shared_ss, ss)
    pltpu.sync_copy(shared_sd.at[pl.ds(base, _PW)], sdv)

    psumr[...] = jnp.zeros((_L,), jnp.float32)
    pcntr[...] = jnp.zeros((_L,), jnp.float32)

    def group_step(grp, carry):
        g0 = base + grp * _L
        xi16 = sx[pl.ds(g0, _L)]
        yi16 = sy[pl.ds(g0, _L)]
        si16 = ss[pl.ds(g0, _L)]
        di16 = sdv[pl.ds(grp * _L, _L)]
        sid16 = jnp.minimum((xi16 * 0.5).astype(jnp.int32), _NSTR - 1)
        lo16 = plsc.load_gather(Cv, [jnp.maximum(sid16 - 1, 0)])
        hi16 = plsc.load_gather(Cv, [sid16 + 2])
        jb016 = lax.shift_right_logical(lo16, 4)
        jb116 = lax.shift_right_logical(hi16 + (_L - 1), 4)

        val16 = jnp.zeros((_L,), jnp.int32)
        for t in range(_L):
            xi = xi16[t]
            yi = yi16[t]
            si = si16[t]

            def win_step(jb, acc):
                sl = pl.ds(jb * _L, _L)
                dx = sx[sl] - xi
                dy = sy[sl] - yi
                d2 = dx * dx + dy * dy
                return jnp.maximum(acc, jnp.where(d2 < _RADIUS2, ss[sl], _NEG))

            acc = lax.fori_loop(jb016[t], jb116[t], win_step,
                                jnp.full((_L,), _NEG, jnp.float32))
            nkeep = plsc.all_reduce_population_count(si >= acc)
            validf = ((nkeep[0] == _L) & (si > _SCORES_TH)).astype(jnp.int32)
            val16 = jnp.where(lane == t, validf, val16)
        vf16 = val16.astype(jnp.float32)
        psumr[...] = psumr[...] + vf16 * di16
        pcntr[...] = pcntr[...] + vf16
        return carry

    lax.fori_loop(0, nvec, group_step, 0)

    pltpu.sync_copy(psumr, out_sum.at[wid, pl.ds(0, _L)])
    pltpu.sync_copy(pcntr, out_cnt.at[wid, pl.ds(0, _L)])


def _sc_call(x, y, s, d):
    mesh = plsc.VectorSubcoreMesh(core_axis_name="c", subcore_axis_name="s",
                                  num_cores=1)
    f = pl.kernel(
        _sc_body,
        out_type=(jax.ShapeDtypeStruct((_NW, _L), jnp.float32),
                  jax.ShapeDtypeStruct((_NW, _L), jnp.float32)),
        mesh=mesh,
        compiler_params=pltpu.CompilerParams(needs_layout_passes=False),
        scratch_types=[
            pltpu.VMEM((_PW,), jnp.float32),
            pltpu.VMEM((_PW,), jnp.float32),
            pltpu.VMEM((_PW,), jnp.float32),
            pltpu.VMEM((_PW,), jnp.float32),
            pltpu.VMEM((_PW,), jnp.int32),
            pltpu.VMEM((_PW,), jnp.int32),
            pltpu.VMEM((_PW,), jnp.int32),
            pltpu.VMEM((_PW,), jnp.int32),
            pltpu.VMEM((_PW // _SCHUNK, _SCHUNK), jnp.int32),
            pltpu.VMEM((_SSZ,), jnp.int32),
            pltpu.VMEM((_NW, _SSZ), jnp.int32),
            pltpu.VMEM((_SSZ,), jnp.int32),
            pltpu.VMEM((_SSZ,), jnp.int32),
            pltpu.VMEM((_CSZ,), jnp.int32),
            pltpu.VMEM((_L,), jnp.int32),
            pltpu.VMEM((_NPAD,), jnp.float32),
            pltpu.VMEM((_NPAD,), jnp.float32),
            pltpu.VMEM((_NPAD,), jnp.float32),
            pltpu.VMEM((_PW,), jnp.float32),
            pltpu.VMEM((_L,), jnp.float32),
            pltpu.VMEM((_L,), jnp.float32),
            pltpu.VMEM_SHARED((_NW, _SSZ), jnp.int32),
            pltpu.VMEM_SHARED((_NPAD,), jnp.float32),
            pltpu.VMEM_SHARED((_NPAD,), jnp.float32),
            pltpu.VMEM_SHARED((_NPAD,), jnp.float32),
            pltpu.VMEM_SHARED((_NPAD,), jnp.float32),
        ],
    )
    return f(x, y, s, d)


def kernel(kpts, scores, dispersity):
    x = kpts[:, 0] * _W
    y = kpts[:, 1] * _H
    pad = _NPAD - _N
    x = jnp.concatenate([x, jnp.full((pad,), _PADX, jnp.float32)])
    y = jnp.concatenate([y, jnp.full((pad,), _PADX, jnp.float32)])
    s = jnp.concatenate([scores, jnp.full((pad,), -1.0, jnp.float32)])
    d = jnp.concatenate([dispersity, jnp.zeros((pad,), jnp.float32)])
    out_sum, out_cnt = _sc_call(x, y, s, d)
    loss_sum = jnp.sum(out_sum)
    cnt = jnp.sum(out_cnt)
    return jnp.where(cnt > 0, loss_sum / jnp.maximum(cnt, 1.0),
                     jnp.float32(0.0))


# speedup vs baseline: 2.4097x; 1.3226x over previous
import jax
import jax.numpy as jnp
from jax import lax
from jax.experimental import pallas as pl
from jax.experimental.pallas import tpu as pltpu
from jax.experimental.pallas import tpu_sc as plsc

_RADIUS2 = 4.0
_SCORES_TH = 0.1
_W = 639.0
_H = 479.0
_N = 5000
_NW = 16
_NPAD = 5120
_PW = _NPAD // _NW
_L = 16
_NSTR = 324
_SSZ = 336
_CSZ = 352
_SCHUNK = 80
_NEG = -3.0e38
_PADX = 1.0e6


def _sc_body(xh, yh, sh, dh, out_sum, out_cnt,
             xv, yv, sv, dv, sidv, occv, lastv, posv1, posv2,
             cnt, allcnt, totv, wpartv, Cv, shuf,
             sx, sy, ss, sdv, psumr, pcntr,
             shared_cnt, shared_sx, shared_sy, shared_ss, shared_sd):
    wid = lax.axis_index("s")
    base = wid * _PW
    lane = lax.iota(jnp.int32, _L)
    nvec = _PW // _L
    ones_i = jnp.ones((_L,), jnp.int32)
    zeros_i = jnp.zeros((_L,), jnp.int32)

    pltpu.sync_copy(xh.at[pl.ds(base, _PW)], xv)
    pltpu.sync_copy(yh.at[pl.ds(base, _PW)], yv)
    pltpu.sync_copy(sh.at[pl.ds(base, _PW)], sv)
    pltpu.sync_copy(dh.at[pl.ds(base, _PW)], dv)

    def sid_step(k, c):
        sl = pl.ds(k * _L, _L)
        sidv[sl] = jnp.minimum((xv[sl] * 0.5).astype(jnp.int32), _NSTR - 1)
        return c

    lax.fori_loop(0, nvec, sid_step, 0)

    for k in range(_SSZ // _L):
        cnt[pl.ds(k * _L, _L)] = zeros_i

    def count_step(k, c):
        sl = pl.ds(k * _L, _L)
        sid = sidv[sl]

        def shift_step(s, oc_fw):
            occ, fwd = oc_fw
            sb = plsc.load_gather(sidv, [k * _L + jnp.maximum(lane - s, 0)])
            sf = plsc.load_gather(sidv,
                                  [k * _L + jnp.minimum(lane + s, _L - 1)])
            occ = occ + ((lane >= s) & (sb == sid)).astype(jnp.int32)
            fwd = fwd + ((lane + s < _L) & (sf == sid)).astype(jnp.int32)
            return occ, fwd

        occ, fwd = lax.fori_loop(1, _L, shift_step, (ones_i, zeros_i))
        last = fwd == 0
        occv[sl] = occ
        lastv[sl] = last.astype(jnp.int32)
        cur = plsc.load_gather(cnt, [sid])
        plsc.store_scatter(cnt, [sid], cur + occ, mask=last)
        return c

    lax.fori_loop(0, nvec, count_step, 0)

    pltpu.sync_copy(cnt, shared_cnt.at[wid])
    plsc.subcore_barrier()
    pltpu.sync_copy(shared_cnt, allcnt)

    for k in range(_SSZ // _L):
        sl = pl.ds(k * _L, _L)
        tot = jnp.zeros((_L,), jnp.int32)
        part = jnp.zeros((_L,), jnp.int32)
        for w in range(_NW):
            row = allcnt[w, sl]
            tot = tot + row
            part = part + row * (jnp.int32(w) < wid).astype(jnp.int32)
        totv[sl] = tot
        wpartv[sl] = part

    npad_i = jnp.full((_L,), _NPAD, jnp.int32)
    for k in range(_SSZ // _L, _CSZ // _L):
        Cv[pl.ds(k * _L, _L)] = npad_i

    def cum_step(k, carry):
        sl = pl.ds(k * _L, _L)
        v = totv[sl]
        p = v
        for s in (1, 2, 4, 8):
            shuf[...] = p
            g = plsc.load_gather(shuf, [jnp.maximum(lane - s, 0)])
            p = p + g * (lane >= s).astype(jnp.int32)
        Cv[sl] = p - v + carry
        return carry + p[_L - 1]

    lax.fori_loop(0, _SSZ // _L, cum_step, jnp.int32(0))

    def curs_step(k, c):
        sl = pl.ds(k * _L, _L)
        wpartv[sl] = Cv[sl] + wpartv[sl]
        return c

    lax.fori_loop(0, _SSZ // _L, curs_step, 0)

    def place_step(k, c):
        sl = pl.ds(k * _L, _L)
        sid = sidv[sl]
        occ = occv[sl]
        last = lastv[sl] == 1
        b = plsc.load_gather(wpartv, [sid])
        posv1[sl] = b + occ - 1
        plsc.store_scatter(wpartv, [sid], b + occ, mask=last)
        return c

    lax.fori_loop(0, nvec, place_step, 0)

    for k in range(nvec):
        posv2[k // (_SCHUNK // _L),
              pl.ds((k % (_SCHUNK // _L)) * _L, _L)] = posv1[pl.ds(k * _L, _L)]

    for c in range(_PW // _SCHUNK):
        sl = pl.ds(c * _SCHUNK, _SCHUNK)
        idx = posv2.at[c]
        pltpu.sync_copy(xv.at[sl], shared_sx.at[idx])
        pltpu.sync_copy(yv.at[sl], shared_sy.at[idx])
        pltpu.sync_copy(sv.at[sl], shared_ss.at[idx])
        pltpu.sync_copy(dv.at[sl], shared_sd.at[idx])
    plsc.subcore_barrier()

    pltpu.sync_copy(shared_sx, sx)
    pltpu.sync_copy(shared_sy, sy)
    pltpu.sync_copy(shared_ss, ss)
    pltpu.sync_copy(shared_sd.at[pl.ds(base, _PW)], sdv)

    psumr[...] = jnp.zeros((_L,), jnp.float32)
    pcntr[...] = jnp.zeros((_L,), jnp.float32)

    def group_step(grp, carry):
        g0 = base + grp * _L
        xi16 = sx[pl.ds(g0, _L)]
        yi16 = sy[pl.ds(g0, _L)]
        si16 = ss[pl.ds(g0, _L)]
        di16 = sdv[pl.ds(grp * _L, _L)]
        sid16 = jnp.minimum((xi16 * 0.5).astype(jnp.int32), _NSTR - 1)
        lo16 = plsc.load_gather(Cv, [jnp.maximum(sid16 - 1, 0)])
        hi16 = plsc.load_gather(Cv, [sid16 + 2])
        jb0 = lax.shift_right_logical(lo16[0], 4)
        jb1 = lax.shift_right_logical(hi16[_L - 1] + (_L - 1), 4)

        def cand_step(jb, acc):
            sl = pl.ds(jb * _L, _L)
            xj16 = sx[sl]
            yj16 = sy[sl]
            sj16 = ss[sl]
            for t in range(_L):
                dx = xi16 - xj16[t]
                dy = yi16 - yj16[t]
                d2 = dx * dx + dy * dy
                acc = jnp.maximum(acc,
                                  jnp.where(d2 < _RADIUS2, sj16[t], _NEG))
            return acc

        acc = lax.fori_loop(jb0, jb1, cand_step,
                            jnp.full((_L,), _NEG, jnp.float32))
        valid = jnp.logical_and(si16 >= acc, si16 > _SCORES_TH)
        vf16 = valid.astype(jnp.float32)
        psumr[...] = psumr[...] + vf16 * di16
        pcntr[...] = pcntr[...] + vf16
        return carry

    lax.fori_loop(0, nvec, group_step, 0)

    pltpu.sync_copy(psumr, out_sum.at[wid, pl.ds(0, _L)])
    pltpu.sync_copy(pcntr, out_cnt.at[wid, pl.ds(0, _L)])


def _sc_call(x, y, s, d):
    mesh = plsc.VectorSubcoreMesh(core_axis_name="c", subcore_axis_name="s",
                                  num_cores=1)
    f = pl.kernel(
        _sc_body,
        out_type=(jax.ShapeDtypeStruct((_NW, _L), jnp.float32),
                  jax.ShapeDtypeStruct((_NW, _L), jnp.float32)),
        mesh=mesh,
        compiler_params=pltpu.CompilerParams(needs_layout_passes=False),
        scratch_types=[
            pltpu.VMEM((_PW,), jnp.float32),
            pltpu.VMEM((_PW,), jnp.float32),
            pltpu.VMEM((_PW,), jnp.float32),
            pltpu.VMEM((_PW,), jnp.float32),
            pltpu.VMEM((_PW,), jnp.int32),
            pltpu.VMEM((_PW,), jnp.int32),
            pltpu.VMEM((_PW,), jnp.int32),
            pltpu.VMEM((_PW,), jnp.int32),
            pltpu.VMEM((_PW // _SCHUNK, _SCHUNK), jnp.int32),
            pltpu.VMEM((_SSZ,), jnp.int32),
            pltpu.VMEM((_NW, _SSZ), jnp.int32),
            pltpu.VMEM((_SSZ,), jnp.int32),
            pltpu.VMEM((_SSZ,), jnp.int32),
            pltpu.VMEM((_CSZ,), jnp.int32),
            pltpu.VMEM((_L,), jnp.int32),
            pltpu.VMEM((_NPAD,), jnp.float32),
            pltpu.VMEM((_NPAD,), jnp.float32),
            pltpu.VMEM((_NPAD,), jnp.float32),
            pltpu.VMEM((_PW,), jnp.float32),
            pltpu.VMEM((_L,), jnp.float32),
            pltpu.VMEM((_L,), jnp.float32),
            pltpu.VMEM_SHARED((_NW, _SSZ), jnp.int32),
            pltpu.VMEM_SHARED((_NPAD,), jnp.float32),
            pltpu.VMEM_SHARED((_NPAD,), jnp.float32),
            pltpu.VMEM_SHARED((_NPAD,), jnp.float32),
            pltpu.VMEM_SHARED((_NPAD,), jnp.float32),
        ],
    )
    return f(x, y, s, d)


def kernel(kpts, scores, dispersity):
    x = kpts[:, 0] * _W
    y = kpts[:, 1] * _H
    pad = _NPAD - _N
    x = jnp.concatenate([x, jnp.full((pad,), _PADX, jnp.float32)])
    y = jnp.concatenate([y, jnp.full((pad,), _PADX, jnp.float32)])
    s = jnp.concatenate([scores, jnp.full((pad,), -1.0, jnp.float32)])
    d = jnp.concatenate([dispersity, jnp.zeros((pad,), jnp.float32)])
    out_sum, out_cnt = _sc_call(x, y, s, d)
    loss_sum = jnp.sum(out_sum)
    cnt = jnp.sum(out_cnt)
    return jnp.where(cnt > 0, loss_sum / jnp.maximum(cnt, 1.0),
                     jnp.float32(0.0))


# speedup vs baseline: 2.5062x; 1.0401x over previous
import jax
import jax.numpy as jnp
from jax import lax
from jax.experimental import pallas as pl
from jax.experimental.pallas import tpu as pltpu
from jax.experimental.pallas import tpu_sc as plsc

_RADIUS2 = 4.0
_SCORES_TH = 0.1
_W = 639.0
_H = 479.0
_N = 5000
_NW = 16
_NWORK = 32
_NPAD = 5120
_PW = _NPAD // _NW
_PB = _NPAD // _NWORK
_L = 16
_NSTR = 324
_SSZ = 336
_CSZ = 352
_SCHUNK = 80
_NEG = -3.0e38
_PADX = 1.0e6


def _sc_body(xh, yh, sh, dh, out_sum, out_cnt,
             xv, yv, sv, dv, sidv, occv, lastv, posv1, posv2,
             cnt, allcnt, totv, wpartv, Cv, shuf,
             sx, sy, ss, sdv, psumr, pcntr, sem,
             shared_cnt, shared_sx, shared_sy, shared_ss, shared_sd):
    wc = lax.axis_index("c")
    wid = lax.axis_index("s")
    rid = wid * 2 + wc
    base = wid * _PW
    lane = lax.iota(jnp.int32, _L)
    nvec = _PW // _L
    ones_i = jnp.ones((_L,), jnp.int32)
    zeros_i = jnp.zeros((_L,), jnp.int32)

    pltpu.sync_copy(xh.at[pl.ds(base, _PW)], xv)
    pltpu.sync_copy(yh.at[pl.ds(base, _PW)], yv)
    pltpu.sync_copy(sh.at[pl.ds(base, _PW)], sv)
    pltpu.sync_copy(dh.at[pl.ds(base, _PW)], dv)

    def sid_step(k, c):
        sl = pl.ds(k * _L, _L)
        sidv[sl] = jnp.minimum((xv[sl] * 0.5).astype(jnp.int32), _NSTR - 1)
        return c

    lax.fori_loop(0, nvec, sid_step, 0)

    for k in range(_SSZ // _L):
        cnt[pl.ds(k * _L, _L)] = zeros_i

    def count_step(k, c):
        sl = pl.ds(k * _L, _L)
        sid = sidv[sl]

        def shift_step(s, oc_fw):
            occ, fwd = oc_fw
            sb = plsc.load_gather(sidv, [k * _L + jnp.maximum(lane - s, 0)])
            sf = plsc.load_gather(sidv,
                                  [k * _L + jnp.minimum(lane + s, _L - 1)])
            occ = occ + ((lane >= s) & (sb == sid)).astype(jnp.int32)
            fwd = fwd + ((lane + s < _L) & (sf == sid)).astype(jnp.int32)
            return occ, fwd

        occ, fwd = lax.fori_loop(1, _L, shift_step, (ones_i, zeros_i))
        last = fwd == 0
        occv[sl] = occ
        lastv[sl] = last.astype(jnp.int32)
        cur = plsc.load_gather(cnt, [sid])
        plsc.store_scatter(cnt, [sid], cur + occ, mask=last)
        return c

    lax.fori_loop(0, nvec, count_step, 0)

    pltpu.sync_copy(cnt, shared_cnt.at[wid])
    plsc.subcore_barrier()
    pltpu.sync_copy(shared_cnt, allcnt)

    for k in range(_SSZ // _L):
        sl = pl.ds(k * _L, _L)
        tot = jnp.zeros((_L,), jnp.int32)
        part = jnp.zeros((_L,), jnp.int32)
        for w in range(_NW):
            row = allcnt[w, sl]
            tot = tot + row
            part = part + row * (jnp.int32(w) < wid).astype(jnp.int32)
        totv[sl] = tot
        wpartv[sl] = part

    npad_i = jnp.full((_L,), _NPAD, jnp.int32)
    for k in range(_SSZ // _L, _CSZ // _L):
        Cv[pl.ds(k * _L, _L)] = npad_i

    def cum_step(k, carry):
        sl = pl.ds(k * _L, _L)
        v = totv[sl]
        p = v
        for s in (1, 2, 4, 8):
            shuf[...] = p
            g = plsc.load_gather(shuf, [jnp.maximum(lane - s, 0)])
            p = p + g * (lane >= s).astype(jnp.int32)
        Cv[sl] = p - v + carry
        return carry + p[_L - 1]

    lax.fori_loop(0, _SSZ // _L, cum_step, jnp.int32(0))

    def curs_step(k, c):
        sl = pl.ds(k * _L, _L)
        wpartv[sl] = Cv[sl] + wpartv[sl]
        return c

    lax.fori_loop(0, _SSZ // _L, curs_step, 0)

    def place_step(k, c):
        sl = pl.ds(k * _L, _L)
        sid = sidv[sl]
        occ = occv[sl]
        last = lastv[sl] == 1
        b = plsc.load_gather(wpartv, [sid])
        posv1[sl] = b + occ - 1
        plsc.store_scatter(wpartv, [sid], b + occ, mask=last)
        return c

    lax.fori_loop(0, nvec, place_step, 0)

    for k in range(nvec):
        posv2[k // (_SCHUNK // _L),
              pl.ds((k % (_SCHUNK // _L)) * _L, _L)] = posv1[pl.ds(k * _L, _L)]

    descs = []
    for c in range(_PW // _SCHUNK):
        sl = pl.ds(c * _SCHUNK, _SCHUNK)
        idx = posv2.at[c]
        descs.append(pltpu.async_copy(xv.at[sl], shared_sx.at[idx], sem))
        descs.append(pltpu.async_copy(yv.at[sl], shared_sy.at[idx], sem))
        descs.append(pltpu.async_copy(sv.at[sl], shared_ss.at[idx], sem))
        descs.append(pltpu.async_copy(dv.at[sl], shared_sd.at[idx], sem))
    for dsc in descs:
        dsc.wait()
    plsc.subcore_barrier()

    bbase = rid * _PB
    pltpu.sync_copy(shared_sx, sx)
    pltpu.sync_copy(shared_sy, sy)
    pltpu.sync_copy(shared_ss, ss)
    pltpu.sync_copy(shared_sd.at[pl.ds(bbase, _PB)], sdv)

    psumr[...] = jnp.zeros((_L,), jnp.float32)
    pcntr[...] = jnp.zeros((_L,), jnp.float32)

    def group_step(grp, carry):
        g0 = bbase + grp * _L
        xi16 = sx[pl.ds(g0, _L)]
        yi16 = sy[pl.ds(g0, _L)]
        si16 = ss[pl.ds(g0, _L)]
        di16 = sdv[pl.ds(grp * _L, _L)]
        sid16 = jnp.minimum((xi16 * 0.5).astype(jnp.int32), _NSTR - 1)
        lo16 = plsc.load_gather(Cv, [jnp.maximum(sid16 - 1, 0)])
        hi16 = plsc.load_gather(Cv, [sid16 + 2])
        jb0 = lax.shift_right_logical(lo16[0], 4)
        jb1 = lax.shift_right_logical(hi16[_L - 1] + (_L - 1), 4)

        def cand_step(jb, acc):
            sl = pl.ds(jb * _L, _L)
            xj16 = sx[sl]
            yj16 = sy[sl]
            sj16 = ss[sl]
            for t in range(_L):
                dx = xi16 - xj16[t]
                dy = yi16 - yj16[t]
                d2 = dx * dx + dy * dy
                acc = jnp.maximum(acc,
                                  jnp.where(d2 < _RADIUS2, sj16[t], _NEG))
            return acc

        acc = lax.fori_loop(jb0, jb1, cand_step,
                            jnp.full((_L,), _NEG, jnp.float32))
        valid = jnp.logical_and(si16 >= acc, si16 > _SCORES_TH)
        vf16 = valid.astype(jnp.float32)
        psumr[...] = psumr[...] + vf16 * di16
        pcntr[...] = pcntr[...] + vf16
        return carry

    lax.fori_loop(0, _PB // _L, group_step, 0)

    pltpu.sync_copy(psumr, out_sum.at[rid, pl.ds(0, _L)])
    pltpu.sync_copy(pcntr, out_cnt.at[rid, pl.ds(0, _L)])


def _sc_call(x, y, s, d):
    mesh = plsc.VectorSubcoreMesh(core_axis_name="c", subcore_axis_name="s",
                                  num_cores=2)
    f = pl.kernel(
        _sc_body,
        out_type=(jax.ShapeDtypeStruct((_NWORK, _L), jnp.float32),
                  jax.ShapeDtypeStruct((_NWORK, _L), jnp.float32)),
        mesh=mesh,
        compiler_params=pltpu.CompilerParams(needs_layout_passes=False),
        scratch_types=[
            pltpu.VMEM((_PW,), jnp.float32),
            pltpu.VMEM((_PW,), jnp.float32),
            pltpu.VMEM((_PW,), jnp.float32),
            pltpu.VMEM((_PW,), jnp.float32),
            pltpu.VMEM((_PW,), jnp.int32),
            pltpu.VMEM((_PW,), jnp.int32),
            pltpu.VMEM((_PW,), jnp.int32),
            pltpu.VMEM((_PW,), jnp.int32),
            pltpu.VMEM((_PW // _SCHUNK, _SCHUNK), jnp.int32),
            pltpu.VMEM((_SSZ,), jnp.int32),
            pltpu.VMEM((_NW, _SSZ), jnp.int32),
            pltpu.VMEM((_SSZ,), jnp.int32),
            pltpu.VMEM((_SSZ,), jnp.int32),
            pltpu.VMEM((_CSZ,), jnp.int32),
            pltpu.VMEM((_L,), jnp.int32),
            pltpu.VMEM((_NPAD,), jnp.float32),
            pltpu.VMEM((_NPAD,), jnp.float32),
            pltpu.VMEM((_NPAD,), jnp.float32),
            pltpu.VMEM((_PB,), jnp.float32),
            pltpu.VMEM((_L,), jnp.float32),
            pltpu.VMEM((_L,), jnp.float32),
            pltpu.SemaphoreType.DMA,
            pltpu.VMEM_SHARED((_NW, _SSZ), jnp.int32),
            pltpu.VMEM_SHARED((_NPAD,), jnp.float32),
            pltpu.VMEM_SHARED((_NPAD,), jnp.float32),
            pltpu.VMEM_SHARED((_NPAD,), jnp.float32),
            pltpu.VMEM_SHARED((_NPAD,), jnp.float32),
        ],
    )
    return f(x, y, s, d)


def kernel(kpts, scores, dispersity):
    x = kpts[:, 0] * _W
    y = kpts[:, 1] * _H
    pad = _NPAD - _N
    x = jnp.concatenate([x, jnp.full((pad,), _PADX, jnp.float32)])
    y = jnp.concatenate([y, jnp.full((pad,), _PADX, jnp.float32)])
    s = jnp.concatenate([scores, jnp.full((pad,), -1.0, jnp.float32)])
    d = jnp.concatenate([dispersity, jnp.zeros((pad,), jnp.float32)])
    out_sum, out_cnt = _sc_call(x, y, s, d)
    loss_sum = jnp.sum(out_sum)
    cnt = jnp.sum(out_cnt)
    return jnp.where(cnt > 0, loss_sum / jnp.maximum(cnt, 1.0),
                     jnp.float32(0.0))


# speedup vs baseline: 2.6382x; 1.0527x over previous
import jax
import jax.numpy as jnp
from jax import lax
from jax.experimental import pallas as pl
from jax.experimental.pallas import tpu as pltpu
from jax.experimental.pallas import tpu_sc as plsc

_RADIUS2 = 4.0
_SCORES_TH = 0.1
_W = 639.0
_H = 479.0
_N = 5000
_NW = 16
_NWORK = 32
_NPAD = 5120
_PW = _NPAD // _NW
_PB = _NPAD // _NWORK
_L = 16
_NSTR = 324
_SSZ = 336
_CSZ = 352
_SCHUNK = 80
_NEG = -3.0e38
_PADX = 1.0e6


def _sc_body(xh, yh, sh, dh, out_sum, out_cnt,
             xv, yv, sv, dv, sidv, occv, lastv, posv1, posv2,
             cnt, allcnt, totv, wpartv, Cv, shuf,
             sx, sy, ss, sdv, psumr, pcntr, sem,
             shared_cnt, shared_sx, shared_sy, shared_ss, shared_sd):
    wc = lax.axis_index("c")
    wid = lax.axis_index("s")
    rid = wid * 2 + wc
    base = wid * _PW
    lane = lax.iota(jnp.int32, _L)
    nvec = _PW // _L
    ones_i = jnp.ones((_L,), jnp.int32)
    zeros_i = jnp.zeros((_L,), jnp.int32)

    pltpu.sync_copy(xh.at[pl.ds(base, _PW)], xv)
    pltpu.sync_copy(yh.at[pl.ds(base, _PW)], yv)
    pltpu.sync_copy(sh.at[pl.ds(base, _PW)], sv)
    pltpu.sync_copy(dh.at[pl.ds(base, _PW)], dv)

    def sid_step(k, c):
        sl = pl.ds(k * _L, _L)
        sidv[sl] = jnp.minimum((xv[sl] * 0.5).astype(jnp.int32), _NSTR - 1)
        return c

    lax.fori_loop(0, nvec, sid_step, 0)

    for k in range(_SSZ // _L):
        cnt[pl.ds(k * _L, _L)] = zeros_i

    def count_step(k, c):
        sl = pl.ds(k * _L, _L)
        sid = sidv[sl]
        occ, last = plsc.scan_count(sid)
        occv[sl] = occ
        lastv[sl] = last.astype(jnp.int32)
        cur = plsc.load_gather(cnt, [sid])
        plsc.store_scatter(cnt, [sid], cur + occ, mask=last)
        return c

    lax.fori_loop(0, nvec, count_step, 0)

    pltpu.sync_copy(cnt, shared_cnt.at[wid])
    plsc.subcore_barrier()
    pltpu.sync_copy(shared_cnt, allcnt)

    for k in range(_SSZ // _L):
        sl = pl.ds(k * _L, _L)
        tot = jnp.zeros((_L,), jnp.int32)
        part = jnp.zeros((_L,), jnp.int32)
        for w in range(_NW):
            row = allcnt[w, sl]
            tot = tot + row
            part = part + row * (jnp.int32(w) < wid).astype(jnp.int32)
        totv[sl] = tot
        wpartv[sl] = part

    npad_i = jnp.full((_L,), _NPAD, jnp.int32)
    for k in range(_SSZ // _L, _CSZ // _L):
        Cv[pl.ds(k * _L, _L)] = npad_i

    def cum_step(k, carry):
        sl = pl.ds(k * _L, _L)
        v = totv[sl]
        p = plsc.cumsum(v)
        Cv[sl] = p - v + carry
        return carry + p[_L - 1]

    lax.fori_loop(0, _SSZ // _L, cum_step, jnp.int32(0))

    def curs_step(k, c):
        sl = pl.ds(k * _L, _L)
        wpartv[sl] = Cv[sl] + wpartv[sl]
        return c

    lax.fori_loop(0, _SSZ // _L, curs_step, 0)

    def place_step(k, c):
        sl = pl.ds(k * _L, _L)
        sid = sidv[sl]
        occ = occv[sl]
        last = lastv[sl] == 1
        b = plsc.load_gather(wpartv, [sid])
        posv1[sl] = b + occ - 1
        plsc.store_scatter(wpartv, [sid], b + occ, mask=last)
        return c

    lax.fori_loop(0, nvec, place_step, 0)

    for k in range(nvec):
        posv2[k // (_SCHUNK // _L),
              pl.ds((k % (_SCHUNK // _L)) * _L, _L)] = posv1[pl.ds(k * _L, _L)]

    descs = []
    for c in range(_PW // _SCHUNK):
        sl = pl.ds(c * _SCHUNK, _SCHUNK)
        idx = posv2.at[c]
        descs.append(pltpu.async_copy(xv.at[sl], shared_sx.at[idx], sem))
        descs.append(pltpu.async_copy(yv.at[sl], shared_sy.at[idx], sem))
        descs.append(pltpu.async_copy(sv.at[sl], shared_ss.at[idx], sem))
        descs.append(pltpu.async_copy(dv.at[sl], shared_sd.at[idx], sem))
    for dsc in descs:
        dsc.wait()
    plsc.subcore_barrier()

    bbase = rid * _PB
    pltpu.sync_copy(shared_sx, sx)
    pltpu.sync_copy(shared_sy, sy)
    pltpu.sync_copy(shared_ss, ss)
    pltpu.sync_copy(shared_sd.at[pl.ds(bbase, _PB)], sdv)

    psumr[...] = jnp.zeros((_L,), jnp.float32)
    pcntr[...] = jnp.zeros((_L,), jnp.float32)

    def group_step(grp, carry):
        g0 = bbase + grp * _L
        xi16 = sx[pl.ds(g0, _L)]
        yi16 = sy[pl.ds(g0, _L)]
        si16 = ss[pl.ds(g0, _L)]
        di16 = sdv[pl.ds(grp * _L, _L)]
        sid16 = jnp.minimum((xi16 * 0.5).astype(jnp.int32), _NSTR - 1)
        lo16 = plsc.load_gather(Cv, [jnp.maximum(sid16 - 1, 0)])
        hi16 = plsc.load_gather(Cv, [sid16 + 2])
        jb0 = lax.shift_right_logical(lo16[0], 4)
        jb1 = lax.shift_right_logical(hi16[_L - 1] + (_L - 1), 4)

        def cand_step(jb, acc):
            sl = pl.ds(jb * _L, _L)
            xj16 = sx[sl]
            yj16 = sy[sl]
            sj16 = ss[sl]
            for t in range(_L):
                dx = xi16 - xj16[t]
                dy = yi16 - yj16[t]
                d2 = dx * dx + dy * dy
                acc = jnp.maximum(acc,
                                  jnp.where(d2 < _RADIUS2, sj16[t], _NEG))
            return acc

        acc = lax.fori_loop(jb0, jb1, cand_step,
                            jnp.full((_L,), _NEG, jnp.float32))
        valid = jnp.logical_and(si16 >= acc, si16 > _SCORES_TH)
        vf16 = valid.astype(jnp.float32)
        psumr[...] = psumr[...] + vf16 * di16
        pcntr[...] = pcntr[...] + vf16
        return carry

    lax.fori_loop(0, _PB // _L, group_step, 0)

    pltpu.sync_copy(psumr, out_sum.at[rid, pl.ds(0, _L)])
    pltpu.sync_copy(pcntr, out_cnt.at[rid, pl.ds(0, _L)])


def _sc_call(x, y, s, d):
    mesh = plsc.VectorSubcoreMesh(core_axis_name="c", subcore_axis_name="s",
                                  num_cores=2)
    f = pl.kernel(
        _sc_body,
        out_type=(jax.ShapeDtypeStruct((_NWORK, _L), jnp.float32),
                  jax.ShapeDtypeStruct((_NWORK, _L), jnp.float32)),
        mesh=mesh,
        compiler_params=pltpu.CompilerParams(needs_layout_passes=False),
        scratch_types=[
            pltpu.VMEM((_PW,), jnp.float32),
            pltpu.VMEM((_PW,), jnp.float32),
            pltpu.VMEM((_PW,), jnp.float32),
            pltpu.VMEM((_PW,), jnp.float32),
            pltpu.VMEM((_PW,), jnp.int32),
            pltpu.VMEM((_PW,), jnp.int32),
            pltpu.VMEM((_PW,), jnp.int32),
            pltpu.VMEM((_PW,), jnp.int32),
            pltpu.VMEM((_PW // _SCHUNK, _SCHUNK), jnp.int32),
            pltpu.VMEM((_SSZ,), jnp.int32),
            pltpu.VMEM((_NW, _SSZ), jnp.int32),
            pltpu.VMEM((_SSZ,), jnp.int32),
            pltpu.VMEM((_SSZ,), jnp.int32),
            pltpu.VMEM((_CSZ,), jnp.int32),
            pltpu.VMEM((_L,), jnp.int32),
            pltpu.VMEM((_NPAD,), jnp.float32),
            pltpu.VMEM((_NPAD,), jnp.float32),
            pltpu.VMEM((_NPAD,), jnp.float32),
            pltpu.VMEM((_PB,), jnp.float32),
            pltpu.VMEM((_L,), jnp.float32),
            pltpu.VMEM((_L,), jnp.float32),
            pltpu.SemaphoreType.DMA,
            pltpu.VMEM_SHARED((_NW, _SSZ), jnp.int32),
            pltpu.VMEM_SHARED((_NPAD,), jnp.float32),
            pltpu.VMEM_SHARED((_NPAD,), jnp.float32),
            pltpu.VMEM_SHARED((_NPAD,), jnp.float32),
            pltpu.VMEM_SHARED((_NPAD,), jnp.float32),
        ],
    )
    return f(x, y, s, d)


def kernel(kpts, scores, dispersity):
    x = kpts[:, 0] * _W
    y = kpts[:, 1] * _H
    pad = _NPAD - _N
    x = jnp.concatenate([x, jnp.full((pad,), _PADX, jnp.float32)])
    y = jnp.concatenate([y, jnp.full((pad,), _PADX, jnp.float32)])
    s = jnp.concatenate([scores, jnp.full((pad,), -1.0, jnp.float32)])
    d = jnp.concatenate([dispersity, jnp.zeros((pad,), jnp.float32)])
    out_sum, out_cnt = _sc_call(x, y, s, d)
    loss_sum = jnp.sum(out_sum)
    cnt = jnp.sum(out_cnt)
    return jnp.where(cnt > 0, loss_sum / jnp.maximum(cnt, 1.0),
                     jnp.float32(0.0))
